# Initial kernel scaffold; baseline (speedup 1.0000x reference)
import functools

import jax
import jax.numpy as jnp
from jax import lax
from jax.experimental import pallas as pl
from jax.experimental.pallas import tpu as pltpu
from jax.experimental.pallas import tpu_sc as plsc

_NDIM = 256
_SDIM = 256
_N = 10000
_E = 160000
_G = 64
_NTYPES = 11

_NC = 2
_NS = 16
_EPT = 10240
_EPAD = _EPT * _NS
_GROUP = 256
_NGROUPS = _EPT // _GROUP
_RPT = _N // _NS
_ACC_ROWS = _N + 16
_BLK = 1000
_NBLK = _N // _BLK


def _make_spmm(compute_deg):
  mesh = plsc.VectorSubcoreMesh(core_axis_name="c", subcore_axis_name="s")
  out_type = [jax.ShapeDtypeStruct((_NC, 2, _N, 128), jnp.float32)]
  scratch = [
      pltpu.VMEM((_GROUP, 128), jnp.float32),
      pltpu.VMEM((_EPT // 128, 128), jnp.int32),
      pltpu.VMEM((_EPT // 128, 128), jnp.int32),
      pltpu.VMEM_SHARED((_ACC_ROWS, 128), jnp.float32),
      pltpu.SemaphoreType.DMA,
  ]
  if compute_deg:
    out_type.append(jax.ShapeDtypeStruct((_NC, _N, 16), jnp.float32))
    scratch += [
        pltpu.VMEM((128, 16), jnp.float32),
        pltpu.VMEM((_RPT, 16), jnp.float32),
        pltpu.VMEM_SHARED((_ACC_ROWS, 16), jnp.float32),
    ]

  def body(t0, t1, gidx_h, sidx_h, zeros_h, zeros16_h, ones_h, *refs):
    if compute_deg:
      (out_s, out_d, rows_v, gi_v, si_v, acc, sem,
       ones_v, dwb_v, dacc) = refs
    else:
      out_s, rows_v, gi_v, si_v, acc, sem = refs
    cid = lax.axis_index("c")
    sid = lax.axis_index("s")

    pltpu.sync_copy(gidx_h.at[cid, pl.ds(sid * (_EPT // 128), _EPT // 128)],
                    gi_v)
    pltpu.sync_copy(sidx_h.at[cid, pl.ds(sid * (_EPT // 128), _EPT // 128)],
                    si_v)
    if compute_deg:
      pltpu.sync_copy(ones_h, ones_v)

    for p in range(2):
      tab = t0 if p == 0 else t1
      zb = rows_v.at[pl.ds(0, 125)]
      pltpu.sync_copy(zeros_h, zb)
      for k in range(5):
        pltpu.sync_copy(zb, acc.at[pl.ds(sid * _RPT + k * 125, 125)])
      if p == 0 and compute_deg:
        pltpu.sync_copy(zeros16_h, dwb_v)
        pltpu.sync_copy(dwb_v, dacc.at[pl.ds(sid * _RPT, _RPT)])
      plsc.subcore_barrier()

      def group(g, carry):
        r0 = g * 2
        d0 = pltpu.async_copy(tab.at[gi_v.at[r0]],
                              rows_v.at[pl.ds(0, 128)], sem)
        d1 = pltpu.async_copy(tab.at[gi_v.at[r0 + 1]],
                              rows_v.at[pl.ds(128, 128)], sem)
        d0.wait()
        d1.wait()
        pltpu.sync_copy(rows_v.at[pl.ds(0, 128)], acc.at[si_v.at[r0]],
                        add=True)
        pltpu.sync_copy(rows_v.at[pl.ds(128, 128)], acc.at[si_v.at[r0 + 1]],
                        add=True)
        if p == 0 and compute_deg:
          pltpu.sync_copy(ones_v, dacc.at[si_v.at[r0]], add=True)
          pltpu.sync_copy(ones_v, dacc.at[si_v.at[r0 + 1]], add=True)
        return carry

      lax.fori_loop(0, _NGROUPS, group, 0)
      plsc.subcore_barrier()

      wb = rows_v.at[pl.ds(0, 125)]
      for k in range(5):
        sl = pl.ds(sid * _RPT + k * 125, 125)
        pltpu.sync_copy(acc.at[sl], wb)
        pltpu.sync_copy(wb, out_s.at[cid, p, sl])
      if p == 0 and compute_deg:
        pltpu.sync_copy(dacc.at[pl.ds(sid * _RPT, _RPT)], dwb_v)
        pltpu.sync_copy(dwb_v, out_d.at[cid, pl.ds(sid * _RPT, _RPT)])

  return pl.kernel(body, out_type=tuple(out_type), mesh=mesh,
                   scratch_types=scratch)


_spmm_deg = _make_spmm(True)
_spmm = _make_spmm(False)


def _embed_body(atts_ref, emb_ref, out_ref):
  a = atts_ref[...]
  oh = (a == lax.broadcasted_iota(jnp.int32, (_BLK, _NTYPES), 1)
        ).astype(jnp.float32)
  h = jnp.dot(oh, emb_ref[...], preferred_element_type=jnp.float32)
  out_ref[0] = h[:, :128]
  out_ref[1] = h[:, 128:]


def _embed(atts2, emb):
  return pl.pallas_call(
      _embed_body,
      grid=(_NBLK,),
      in_specs=[
          pl.BlockSpec((_BLK, 1), lambda i: (i, 0)),
          pl.BlockSpec((_NTYPES, _NDIM), lambda i: (0, 0)),
      ],
      out_specs=pl.BlockSpec((2, _BLK, 128), lambda i: (0, i, 0)),
      out_shape=jax.ShapeDtypeStruct((2, _N, 128), jnp.float32),
  )(atts2, emb)


def _layer_body(sin_ref, sout_ref, h_ref, din_ref, dout_ref,
                wf_ref, bf_ref, wr_ref, br_ref,
                wih_ref, bih_ref, whh_ref, bhh_ref, out_ref):
  h = jnp.concatenate([h_ref[0], h_ref[1]], axis=1)
  din = din_ref[...]
  dout = dout_ref[...]
  dot = functools.partial(jnp.dot, preferred_element_type=jnp.float32)
  aggr = (dot(sin_ref[0], wf_ref[0:128])
          + dot(sin_ref[1], wf_ref[128:256])
          + dot(sout_ref[0], wr_ref[0:128])
          + dot(sout_ref[1], wr_ref[128:256])
          + din * dot(h, wf_ref[256:512])
          + dout * dot(h, wr_ref[256:512])
          + din * bf_ref[...] + dout * br_ref[...])
  gi = dot(aggr, wih_ref[...]) + bih_ref[...]
  gh = dot(h, whh_ref[...]) + bhh_ref[...]
  r = jax.nn.sigmoid(gi[:, 0:256] + gh[:, 0:256])
  z = jax.nn.sigmoid(gi[:, 256:512] + gh[:, 256:512])
  n = jnp.tanh(gi[:, 512:768] + r * gh[:, 512:768])
  hn = (1.0 - z) * n + z * h
  out_ref[0] = hn[:, 0:128]
  out_ref[1] = hn[:, 128:256]


def _layer(sin, sout, h, din, dout, wf, bf, wr, br, wih, bih, whh, bhh):
  full = lambda shape: pl.BlockSpec(shape, lambda i: tuple(0 for _ in shape))
  blk3 = pl.BlockSpec((2, _BLK, 128), lambda i: (0, i, 0))
  return pl.pallas_call(
      _layer_body,
      grid=(_NBLK,),
      in_specs=[
          blk3, blk3, blk3,
          pl.BlockSpec((_BLK, 1), lambda i: (i, 0)),
          pl.BlockSpec((_BLK, 1), lambda i: (i, 0)),
          full((2 * _NDIM, 2 * _NDIM)), full((1, 2 * _NDIM)),
          full((2 * _NDIM, 2 * _NDIM)), full((1, 2 * _NDIM)),
          full((2 * _NDIM, 3 * _NDIM)), full((1, 3 * _NDIM)),
          full((_NDIM, 3 * _NDIM)), full((1, 3 * _NDIM)),
      ],
      out_specs=pl.BlockSpec((2, _BLK, 128), lambda i: (0, i, 0)),
      out_shape=jax.ShapeDtypeStruct((2, _N, 128), jnp.float32),
  )(sin, sout, h, din, dout, wf, bf, wr, br, wih, bih, whh, bhh)


def _pool_body(h_ref, batch_ref,
               mfw_ref, mfb_ref, mgw_ref, mgb_ref,
               vfw_ref, vfb_ref, vgw_ref, vgb_ref,
               mout_ref, vout_ref):
  i = pl.program_id(0)

  @pl.when(i == 0)
  def _():
    mout_ref[...] = jnp.zeros_like(mout_ref)
    vout_ref[...] = jnp.zeros_like(vout_ref)

  h = jnp.concatenate([h_ref[0], h_ref[1]], axis=1)
  oh = (batch_ref[...] == lax.broadcasted_iota(jnp.int32, (_BLK, _G), 1)
        ).astype(jnp.float32)
  dot = functools.partial(jnp.dot, preferred_element_type=jnp.float32)
  for fw, fb, gw, gb, out in (
      (mfw_ref, mfb_ref, mgw_ref, mgb_ref, mout_ref),
      (vfw_ref, vfb_ref, vgw_ref, vgb_ref, vout_ref)):
    hv = dot(h, fw[...]) + fb[...]
    g = jax.nn.sigmoid(dot(h, gw[...]) + gb[...])
    out[...] += lax.dot_general(oh, hv * g, (((0,), (0,)), ((), ())),
                                preferred_element_type=jnp.float32)


def _pool(h, batch2, mfw, mfb, mgw, mgb, vfw, vfb, vgw, vgb):
  full = lambda shape: pl.BlockSpec(shape, lambda i: tuple(0 for _ in shape))
  return pl.pallas_call(
      _pool_body,
      grid=(_NBLK,),
      in_specs=[
          pl.BlockSpec((2, _BLK, 128), lambda i: (0, i, 0)),
          pl.BlockSpec((_BLK, 1), lambda i: (i, 0)),
          full((_NDIM, _SDIM)), full((1, _SDIM)),
          full((_NDIM, 1)), full((1, 1)),
          full((_NDIM, _SDIM)), full((1, _SDIM)),
          full((_NDIM, 1)), full((1, 1)),
      ],
      out_specs=[
          pl.BlockSpec((_G, _SDIM), lambda i: (0, 0)),
          pl.BlockSpec((_G, _SDIM), lambda i: (0, 0)),
      ],
      out_shape=[
          jax.ShapeDtypeStruct((_G, _SDIM), jnp.float32),
          jax.ShapeDtypeStruct((_G, _SDIM), jnp.float32),
      ],
  )(h, batch2, mfw, mfb, mgw, mgb, vfw, vfb, vgw, vgb)


def kernel(edge_index, node_atts, batch, params):
  p = params
  src = edge_index[0].astype(jnp.int32)
  dst = edge_index[1].astype(jnp.int32)

  npad = _EPAD - _E
  ar = jnp.arange(npad, dtype=jnp.int32)
  pad_g = (ar * 97) % _N
  pad_s = _N + (ar % 16)
  gidx = jnp.stack([jnp.concatenate([src, pad_g]),
                    jnp.concatenate([dst, pad_g])]).reshape(
                        2, _EPAD // 128, 128)
  sidx = jnp.stack([jnp.concatenate([dst, pad_s]),
                    jnp.concatenate([src, pad_s])]).reshape(
                        2, _EPAD // 128, 128)
  zeros_h = jnp.zeros((125, 128), jnp.float32)
  zeros16_h = jnp.zeros((_RPT, 16), jnp.float32)
  ones_h = jnp.ones((128, 16), jnp.float32)

  atts2 = node_atts.astype(jnp.int32).reshape(_N, 1)
  batch2 = batch.astype(jnp.int32).reshape(_N, 1)
  r2 = lambda b: b.reshape(1, -1)

  h = _embed(atts2, p['emb'])

  out_s, out_d = _spmm_deg(h[0], h[1], gidx, sidx, zeros_h, zeros16_h,
                           ones_h)
  din = out_d[0, :, 0:1]
  dout = out_d[1, :, 0:1]
  h = _layer(out_s[0], out_s[1], h, din, dout,
             p['msg_W_0'], r2(p['msg_b_0']), p['msgr_W_0'], r2(p['msgr_b_0']),
             p['W_ih_0'], r2(p['b_ih_0']), p['W_hh_0'], r2(p['b_hh_0']))

  out_s2 = _spmm(h[0], h[1], gidx, sidx, zeros_h, zeros16_h, ones_h)
  if isinstance(out_s2, (list, tuple)):
    out_s2 = out_s2[0]
  h = _layer(out_s2[0], out_s2[1], h, din, dout,
             p['msg_W_1'], r2(p['msg_b_1']), p['msgr_W_1'], r2(p['msgr_b_1']),
             p['W_ih_1'], r2(p['b_ih_1']), p['W_hh_1'], r2(p['b_hh_1']))

  return tuple(_pool(h, batch2,
                     p['mean_fm_W'], r2(p['mean_fm_b']),
                     p['mean_gm_W'], r2(p['mean_gm_b']),
                     p['var_fm_W'], r2(p['var_fm_b']),
                     p['var_gm_W'], r2(p['var_gm_b'])))


# speedup vs baseline: 10.1770x; 10.1770x over previous
import functools

import jax
import jax.numpy as jnp
from jax import lax
from jax.experimental import pallas as pl
from jax.experimental.pallas import tpu as pltpu
from jax.experimental.pallas import tpu_sc as plsc

_NDIM = 256
_SDIM = 256
_N = 10000
_E = 160000
_G = 64
_NTYPES = 11

_NC = 2
_NS = 16
_EPT = 10240
_EPAD = _EPT * _NS
_GROUP = 128
_NGROUPS = _EPT // _GROUP
_ACC_ROWS = 10112
_RPT = _ACC_ROWS // _NS
_WB_CHUNKS = (128, 128, 128, 128, 120)
_BLK = 1000
_NBLK = _N // _BLK


def _make_spmm():
  mesh = plsc.VectorSubcoreMesh(core_axis_name="c", subcore_axis_name="s",
                                num_cores=_NC, num_subcores=_NS)
  out_type = jax.ShapeDtypeStruct((_NC, 2, _ACC_ROWS, 128), jnp.float32)
  scratch = [
      pltpu.VMEM((_GROUP, 128), jnp.float32),
      pltpu.VMEM((_EPT // 256, 128), jnp.int32),
      pltpu.VMEM((_EPT // 256, 128), jnp.int32),
      pltpu.VMEM_SHARED((_ACC_ROWS, 128), jnp.float32),
      pltpu.SemaphoreType.DMA,
  ]

  def body(t0, t1, gidx_h, sidx_h, zeros_h, out_s, rows_v, gi_v, si_v, acc,
           sem):
    cid = lax.axis_index("c")
    sid = lax.axis_index("s")

    for p in range(2):
      tab = t0 if p == 0 else t1
      pltpu.sync_copy(zeros_h, rows_v.at[pl.ds(0, 128)])
      off = 0
      for c in _WB_CHUNKS:
        pltpu.sync_copy(rows_v.at[pl.ds(0, c)],
                        acc.at[pl.ds(sid * _RPT + off, c)])
        off += c
      plsc.subcore_barrier()

      def group(g, carry):
        pltpu.async_copy(tab.at[gi_v.at[g]], rows_v, sem).wait()
        pltpu.sync_copy(rows_v, acc.at[si_v.at[g]], add=True)
        return carry

      nh = _NGROUPS // 2
      for half in range(2):
        row0 = sid * (_EPT // 128) + half * nh
        pltpu.sync_copy(gidx_h.at[cid, pl.ds(row0, nh)], gi_v)
        pltpu.sync_copy(sidx_h.at[cid, pl.ds(row0, nh)], si_v)
        lax.fori_loop(0, nh, group, 0)
      plsc.subcore_barrier()

      off = 0
      for c in _WB_CHUNKS:
        sl = pl.ds(sid * _RPT + off, c)
        pltpu.sync_copy(acc.at[sl], rows_v.at[pl.ds(0, c)])
        pltpu.sync_copy(rows_v.at[pl.ds(0, c)], out_s.at[cid, p, sl])
        off += c

  return pl.kernel(body, out_type=out_type, mesh=mesh,
                   scratch_types=scratch)


def _make_deg():
  mesh = plsc.VectorSubcoreMesh(core_axis_name="c", subcore_axis_name="s",
                                num_cores=_NC, num_subcores=_NS)
  out_type = jax.ShapeDtypeStruct((_NC, _ACC_ROWS, 128), jnp.float32)
  scratch = [
      pltpu.VMEM((128, 128), jnp.float32),
      pltpu.VMEM((_EPT // 128, 128), jnp.int32),
      pltpu.VMEM_SHARED((_ACC_ROWS, 128), jnp.float32),
  ]

  def body(sidx_h, ones_h, zeros_h, out_d, ones_v, si_v, dacc):
    cid = lax.axis_index("c")
    sid = lax.axis_index("s")
    pltpu.sync_copy(sidx_h.at[cid, pl.ds(sid * (_EPT // 128), _EPT // 128)],
                    si_v)
    pltpu.sync_copy(zeros_h, ones_v)
    off = 0
    for c in _WB_CHUNKS:
      pltpu.sync_copy(ones_v.at[pl.ds(0, c)],
                      dacc.at[pl.ds(sid * _RPT + off, c)])
      off += c
    pltpu.sync_copy(ones_h, ones_v)
    plsc.subcore_barrier()

    def group(g, carry):
      pltpu.sync_copy(ones_v, dacc.at[si_v.at[g]], add=True)
      return carry

    lax.fori_loop(0, _NGROUPS, group, 0)
    plsc.subcore_barrier()

    off = 0
    for c in _WB_CHUNKS:
      sl = pl.ds(sid * _RPT + off, c)
      pltpu.sync_copy(dacc.at[sl], ones_v.at[pl.ds(0, c)])
      pltpu.sync_copy(ones_v.at[pl.ds(0, c)], out_d.at[cid, sl])
      off += c

  return pl.kernel(body, out_type=out_type, mesh=mesh,
                   scratch_types=scratch)


@functools.lru_cache(maxsize=None)
def _get_spmm():
  return _make_spmm()


@functools.lru_cache(maxsize=None)
def _get_deg():
  return _make_deg()


def _embed_body(atts_ref, emb_ref, out_ref):
  a = atts_ref[...]
  oh = (a == lax.broadcasted_iota(jnp.int32, (_BLK, _NTYPES), 1)
        ).astype(jnp.float32)
  h = jnp.dot(oh, emb_ref[...], preferred_element_type=jnp.float32)
  out_ref[0] = h[:, :128]
  out_ref[1] = h[:, 128:]


def _embed(atts2, emb):
  return pl.pallas_call(
      _embed_body,
      grid=(_NBLK,),
      in_specs=[
          pl.BlockSpec((_BLK, 1), lambda i: (i, 0)),
          pl.BlockSpec((_NTYPES, _NDIM), lambda i: (0, 0)),
      ],
      out_specs=pl.BlockSpec((2, _BLK, 128), lambda i: (0, i, 0)),
      out_shape=jax.ShapeDtypeStruct((2, _N, 128), jnp.float32),
  )(atts2, emb)


def _layer_body(sin_ref, sout_ref, h_ref, din_ref, dout_ref,
                wf_ref, bf_ref, wr_ref, br_ref,
                wih_ref, bih_ref, whh_ref, bhh_ref, out_ref):
  h = jnp.concatenate([h_ref[0], h_ref[1]], axis=1)
  din = din_ref[...]
  dout = dout_ref[...]
  dot = functools.partial(jnp.dot, preferred_element_type=jnp.float32)
  aggr = (dot(sin_ref[0], wf_ref[0:128])
          + dot(sin_ref[1], wf_ref[128:256])
          + dot(sout_ref[0], wr_ref[0:128])
          + dot(sout_ref[1], wr_ref[128:256])
          + din * dot(h, wf_ref[256:512])
          + dout * dot(h, wr_ref[256:512])
          + din * bf_ref[...] + dout * br_ref[...])
  gi = dot(aggr, wih_ref[...]) + bih_ref[...]
  gh = dot(h, whh_ref[...]) + bhh_ref[...]
  r = jax.nn.sigmoid(gi[:, 0:256] + gh[:, 0:256])
  z = jax.nn.sigmoid(gi[:, 256:512] + gh[:, 256:512])
  n = jnp.tanh(gi[:, 512:768] + r * gh[:, 512:768])
  hn = (1.0 - z) * n + z * h
  out_ref[0] = hn[:, 0:128]
  out_ref[1] = hn[:, 128:256]


def _layer(sin, sout, h, din, dout, wf, bf, wr, br, wih, bih, whh, bhh):
  full = lambda shape: pl.BlockSpec(shape, lambda i: tuple(0 for _ in shape))
  blk3 = pl.BlockSpec((2, _BLK, 128), lambda i: (0, i, 0))
  return pl.pallas_call(
      _layer_body,
      grid=(_NBLK,),
      in_specs=[
          blk3, blk3, blk3,
          pl.BlockSpec((_BLK, 1), lambda i: (i, 0)),
          pl.BlockSpec((_BLK, 1), lambda i: (i, 0)),
          full((2 * _NDIM, 2 * _NDIM)), full((1, 2 * _NDIM)),
          full((2 * _NDIM, 2 * _NDIM)), full((1, 2 * _NDIM)),
          full((2 * _NDIM, 3 * _NDIM)), full((1, 3 * _NDIM)),
          full((_NDIM, 3 * _NDIM)), full((1, 3 * _NDIM)),
      ],
      out_specs=pl.BlockSpec((2, _BLK, 128), lambda i: (0, i, 0)),
      out_shape=jax.ShapeDtypeStruct((2, _N, 128), jnp.float32),
  )(sin, sout, h, din, dout, wf, bf, wr, br, wih, bih, whh, bhh)


def _pool_body(h_ref, batch_ref,
               mfw_ref, mfb_ref, mgw_ref, mgb_ref,
               vfw_ref, vfb_ref, vgw_ref, vgb_ref,
               mout_ref, vout_ref):
  i = pl.program_id(0)

  @pl.when(i == 0)
  def _():
    mout_ref[...] = jnp.zeros_like(mout_ref)
    vout_ref[...] = jnp.zeros_like(vout_ref)

  h = jnp.concatenate([h_ref[0], h_ref[1]], axis=1)
  oh = (batch_ref[...] == lax.broadcasted_iota(jnp.int32, (_BLK, _G), 1)
        ).astype(jnp.float32)
  dot = functools.partial(jnp.dot, preferred_element_type=jnp.float32)
  for fw, fb, gw, gb, out in (
      (mfw_ref, mfb_ref, mgw_ref, mgb_ref, mout_ref),
      (vfw_ref, vfb_ref, vgw_ref, vgb_ref, vout_ref)):
    hv = dot(h, fw[...]) + fb[...]
    g = jax.nn.sigmoid(dot(h, gw[...]) + gb[...])
    out[...] += lax.dot_general(oh, hv * g, (((0,), (0,)), ((), ())),
                                preferred_element_type=jnp.float32)


def _pool(h, batch2, mfw, mfb, mgw, mgb, vfw, vfb, vgw, vgb):
  full = lambda shape: pl.BlockSpec(shape, lambda i: tuple(0 for _ in shape))
  return pl.pallas_call(
      _pool_body,
      grid=(_NBLK,),
      in_specs=[
          pl.BlockSpec((2, _BLK, 128), lambda i: (0, i, 0)),
          pl.BlockSpec((_BLK, 1), lambda i: (i, 0)),
          full((_NDIM, _SDIM)), full((1, _SDIM)),
          full((_NDIM, 1)), full((1, 1)),
          full((_NDIM, _SDIM)), full((1, _SDIM)),
          full((_NDIM, 1)), full((1, 1)),
      ],
      out_specs=[
          pl.BlockSpec((_G, _SDIM), lambda i: (0, 0)),
          pl.BlockSpec((_G, _SDIM), lambda i: (0, 0)),
      ],
      out_shape=[
          jax.ShapeDtypeStruct((_G, _SDIM), jnp.float32),
          jax.ShapeDtypeStruct((_G, _SDIM), jnp.float32),
      ],
  )(h, batch2, mfw, mfb, mgw, mgb, vfw, vfb, vgw, vgb)


def kernel(edge_index, node_atts, batch, params):
  p = params
  src = edge_index[0].astype(jnp.int32)
  dst = edge_index[1].astype(jnp.int32)

  npad = _EPAD - _E
  ar = jnp.arange(npad, dtype=jnp.int32)
  pad_g = (ar * 97) % _N
  pad_s = _N + (ar % 16)
  gidx = jnp.stack([jnp.concatenate([src, pad_g]),
                    jnp.concatenate([dst, pad_g])]).reshape(
                        2, _EPAD // 128, 128)
  sidx = jnp.stack([jnp.concatenate([dst, pad_s]),
                    jnp.concatenate([src, pad_s])]).reshape(
                        2, _EPAD // 128, 128)
  zeros_h = jnp.zeros((128, 128), jnp.float32)

  atts2 = node_atts.astype(jnp.int32).reshape(_N, 1)
  batch2 = batch.astype(jnp.int32).reshape(_N, 1)
  r2 = lambda b: b.reshape(1, -1)

  h = _embed(atts2, p['emb'])

  ones128_h = jnp.ones((128, 128), jnp.float32)
  out_d = _get_deg()(sidx, ones128_h, zeros_h)
  out_s = _get_spmm()(h[0], h[1], gidx, sidx, zeros_h)
  din = out_d[0, :_N, 0:1]
  dout = out_d[1, :_N, 0:1]
  h = _layer(out_s[0][:, :_N], out_s[1][:, :_N], h, din, dout,
             p['msg_W_0'], r2(p['msg_b_0']), p['msgr_W_0'], r2(p['msgr_b_0']),
             p['W_ih_0'], r2(p['b_ih_0']), p['W_hh_0'], r2(p['b_hh_0']))

  out_s2 = _get_spmm()(h[0], h[1], gidx, sidx, zeros_h)
  h = _layer(out_s2[0][:, :_N], out_s2[1][:, :_N], h, din, dout,
             p['msg_W_1'], r2(p['msg_b_1']), p['msgr_W_1'], r2(p['msgr_b_1']),
             p['W_ih_1'], r2(p['b_ih_1']), p['W_hh_1'], r2(p['b_hh_1']))

  return tuple(_pool(h, batch2,
                     p['mean_fm_W'], r2(p['mean_fm_b']),
                     p['mean_gm_W'], r2(p['mean_gm_b']),
                     p['var_fm_W'], r2(p['var_fm_b']),
                     p['var_gm_W'], r2(p['var_gm_b'])))


# speedup vs baseline: 10.9743x; 1.0783x over previous
import functools

import jax
import jax.numpy as jnp
from jax import lax
from jax.experimental import pallas as pl
from jax.experimental.pallas import tpu as pltpu
from jax.experimental.pallas import tpu_sc as plsc

_NDIM = 256
_SDIM = 256
_N = 10000
_E = 160000
_G = 64
_NTYPES = 11

_NC = 2
_NS = 16
_EPT = 10240
_EPAD = _EPT * _NS
_GROUP = 128
_NGROUPS = _EPT // _GROUP
_ACC_ROWS = 10112
_RPT = _ACC_ROWS // _NS
_WB_CHUNKS = (128, 128, 128, 128, 120)
_BLK = 1000
_NBLK = _N // _BLK


def _make_spmm():
  mesh = plsc.VectorSubcoreMesh(core_axis_name="c", subcore_axis_name="s",
                                num_cores=_NC, num_subcores=_NS)
  out_type = jax.ShapeDtypeStruct((_NC, 2, _ACC_ROWS, 128), jnp.float32)
  nstage = 8
  nchunks = (_EPT // 128) // nstage
  scratch = [
      pltpu.VMEM((2, 128, 128), jnp.float32),
      pltpu.VMEM((nstage, 128), jnp.int32),
      pltpu.VMEM((nstage, 128), jnp.int32),
      pltpu.VMEM_SHARED((_ACC_ROWS, 128), jnp.float32),
      pltpu.SemaphoreType.DMA,
      pltpu.SemaphoreType.DMA,
      pltpu.SemaphoreType.DMA,
      pltpu.SemaphoreType.DMA,
  ]

  def body(t0, t1, gidx_h, sidx_h, zeros_h, out_s, rows_v, gi_v, si_v, acc,
           sg0, sg1, ss0, ss1):
    cid = lax.axis_index("c")
    sid = lax.axis_index("s")

    for p in range(2):
      tab = t0 if p == 0 else t1
      pltpu.sync_copy(zeros_h, rows_v.at[0])
      off = 0
      for c in _WB_CHUNKS:
        pltpu.sync_copy(rows_v.at[0, pl.ds(0, c)],
                        acc.at[pl.ds(sid * _RPT + off, c)])
        off += c
      plsc.subcore_barrier()

      def pair(i, carry):
        d0 = pltpu.async_copy(tab.at[gi_v.at[2 * i]], rows_v.at[0], sg0)
        d1 = pltpu.async_copy(tab.at[gi_v.at[2 * i + 1]], rows_v.at[1], sg1)
        d0.wait()
        s0 = pltpu.async_copy(rows_v.at[0], acc.at[si_v.at[2 * i]], ss0,
                              add=True)
        d1.wait()
        s1 = pltpu.async_copy(rows_v.at[1], acc.at[si_v.at[2 * i + 1]], ss1,
                              add=True)
        s0.wait()
        s1.wait()
        return carry

      for chunk in range(nchunks):
        row0 = sid * (_EPT // 128) + chunk * nstage
        pltpu.sync_copy(gidx_h.at[cid, pl.ds(row0, nstage)], gi_v)
        pltpu.sync_copy(sidx_h.at[cid, pl.ds(row0, nstage)], si_v)
        lax.fori_loop(0, nstage // 2, pair, 0)
      plsc.subcore_barrier()

      off = 0
      for c in _WB_CHUNKS:
        sl = pl.ds(sid * _RPT + off, c)
        pltpu.sync_copy(acc.at[sl], rows_v.at[0, pl.ds(0, c)])
        pltpu.sync_copy(rows_v.at[0, pl.ds(0, c)], out_s.at[cid, p, sl])
        off += c

  return pl.kernel(body, out_type=out_type, mesh=mesh,
                   scratch_types=scratch)


def _make_deg():
  mesh = plsc.VectorSubcoreMesh(core_axis_name="c", subcore_axis_name="s",
                                num_cores=_NC, num_subcores=_NS)
  out_type = jax.ShapeDtypeStruct((_NC, _ACC_ROWS, 128), jnp.float32)
  scratch = [
      pltpu.VMEM((128, 128), jnp.float32),
      pltpu.VMEM((_EPT // 128, 128), jnp.int32),
      pltpu.VMEM_SHARED((_ACC_ROWS, 128), jnp.float32),
      pltpu.SemaphoreType.DMA,
  ]

  def body(sidx_h, ones_h, zeros_h, out_d, ones_v, si_v, dacc, sem):
    cid = lax.axis_index("c")
    sid = lax.axis_index("s")
    pltpu.sync_copy(sidx_h.at[cid, pl.ds(sid * (_EPT // 128), _EPT // 128)],
                    si_v)
    pltpu.sync_copy(zeros_h, ones_v)
    off = 0
    for c in _WB_CHUNKS:
      pltpu.sync_copy(ones_v.at[pl.ds(0, c)],
                      dacc.at[pl.ds(sid * _RPT + off, c)])
      off += c
    pltpu.sync_copy(ones_h, ones_v)
    plsc.subcore_barrier()

    def group(i, carry):
      ds = [pltpu.async_copy(ones_v, dacc.at[si_v.at[4 * i + j]], sem,
                             add=True) for j in range(4)]
      for d in ds:
        d.wait()
      return carry

    lax.fori_loop(0, _NGROUPS // 4, group, 0)
    plsc.subcore_barrier()

    off = 0
    for c in _WB_CHUNKS:
      sl = pl.ds(sid * _RPT + off, c)
      pltpu.sync_copy(dacc.at[sl], ones_v.at[pl.ds(0, c)])
      pltpu.sync_copy(ones_v.at[pl.ds(0, c)], out_d.at[cid, sl])
      off += c

  return pl.kernel(body, out_type=out_type, mesh=mesh,
                   scratch_types=scratch)


@functools.lru_cache(maxsize=None)
def _get_spmm():
  return _make_spmm()


@functools.lru_cache(maxsize=None)
def _get_deg():
  return _make_deg()


def _embed_body(atts_ref, emb_ref, out_ref):
  a = atts_ref[...]
  oh = (a == lax.broadcasted_iota(jnp.int32, (_BLK, _NTYPES), 1)
        ).astype(jnp.float32)
  h = jnp.dot(oh, emb_ref[...], preferred_element_type=jnp.float32)
  out_ref[0] = h[:, :128]
  out_ref[1] = h[:, 128:]


def _embed(atts2, emb):
  return pl.pallas_call(
      _embed_body,
      grid=(_NBLK,),
      in_specs=[
          pl.BlockSpec((_BLK, 1), lambda i: (i, 0)),
          pl.BlockSpec((_NTYPES, _NDIM), lambda i: (0, 0)),
      ],
      out_specs=pl.BlockSpec((2, _BLK, 128), lambda i: (0, i, 0)),
      out_shape=jax.ShapeDtypeStruct((2, _N, 128), jnp.float32),
  )(atts2, emb)


def _layer_body(sin_ref, sout_ref, h_ref, din_ref, dout_ref,
                wf_ref, bf_ref, wr_ref, br_ref,
                wih_ref, bih_ref, whh_ref, bhh_ref, out_ref):
  h = jnp.concatenate([h_ref[0], h_ref[1]], axis=1)
  din = din_ref[...]
  dout = dout_ref[...]
  dot = functools.partial(jnp.dot, preferred_element_type=jnp.float32)
  aggr = (dot(sin_ref[0], wf_ref[0:128])
          + dot(sin_ref[1], wf_ref[128:256])
          + dot(sout_ref[0], wr_ref[0:128])
          + dot(sout_ref[1], wr_ref[128:256])
          + din * dot(h, wf_ref[256:512])
          + dout * dot(h, wr_ref[256:512])
          + din * bf_ref[...] + dout * br_ref[...])
  gi = dot(aggr, wih_ref[...]) + bih_ref[...]
  gh = dot(h, whh_ref[...]) + bhh_ref[...]
  r = jax.nn.sigmoid(gi[:, 0:256] + gh[:, 0:256])
  z = jax.nn.sigmoid(gi[:, 256:512] + gh[:, 256:512])
  n = jnp.tanh(gi[:, 512:768] + r * gh[:, 512:768])
  hn = (1.0 - z) * n + z * h
  out_ref[0] = hn[:, 0:128]
  out_ref[1] = hn[:, 128:256]


def _layer(sin, sout, h, din, dout, wf, bf, wr, br, wih, bih, whh, bhh):
  full = lambda shape: pl.BlockSpec(shape, lambda i: tuple(0 for _ in shape))
  blk3 = pl.BlockSpec((2, _BLK, 128), lambda i: (0, i, 0))
  return pl.pallas_call(
      _layer_body,
      grid=(_NBLK,),
      in_specs=[
          blk3, blk3, blk3,
          pl.BlockSpec((_BLK, 1), lambda i: (i, 0)),
          pl.BlockSpec((_BLK, 1), lambda i: (i, 0)),
          full((2 * _NDIM, 2 * _NDIM)), full((1, 2 * _NDIM)),
          full((2 * _NDIM, 2 * _NDIM)), full((1, 2 * _NDIM)),
          full((2 * _NDIM, 3 * _NDIM)), full((1, 3 * _NDIM)),
          full((_NDIM, 3 * _NDIM)), full((1, 3 * _NDIM)),
      ],
      out_specs=pl.BlockSpec((2, _BLK, 128), lambda i: (0, i, 0)),
      out_shape=jax.ShapeDtypeStruct((2, _N, 128), jnp.float32),
  )(sin, sout, h, din, dout, wf, bf, wr, br, wih, bih, whh, bhh)


def _pool_body(h_ref, batch_ref,
               mfw_ref, mfb_ref, mgw_ref, mgb_ref,
               vfw_ref, vfb_ref, vgw_ref, vgb_ref,
               mout_ref, vout_ref):
  i = pl.program_id(0)

  @pl.when(i == 0)
  def _():
    mout_ref[...] = jnp.zeros_like(mout_ref)
    vout_ref[...] = jnp.zeros_like(vout_ref)

  h = jnp.concatenate([h_ref[0], h_ref[1]], axis=1)
  oh = (batch_ref[...] == lax.broadcasted_iota(jnp.int32, (_BLK, _G), 1)
        ).astype(jnp.float32)
  dot = functools.partial(jnp.dot, preferred_element_type=jnp.float32)
  for fw, fb, gw, gb, out in (
      (mfw_ref, mfb_ref, mgw_ref, mgb_ref, mout_ref),
      (vfw_ref, vfb_ref, vgw_ref, vgb_ref, vout_ref)):
    hv = dot(h, fw[...]) + fb[...]
    g = jax.nn.sigmoid(dot(h, gw[...]) + gb[...])
    out[...] += lax.dot_general(oh, hv * g, (((0,), (0,)), ((), ())),
                                preferred_element_type=jnp.float32)


def _pool(h, batch2, mfw, mfb, mgw, mgb, vfw, vfb, vgw, vgb):
  full = lambda shape: pl.BlockSpec(shape, lambda i: tuple(0 for _ in shape))
  return pl.pallas_call(
      _pool_body,
      grid=(_NBLK,),
      in_specs=[
          pl.BlockSpec((2, _BLK, 128), lambda i: (0, i, 0)),
          pl.BlockSpec((_BLK, 1), lambda i: (i, 0)),
          full((_NDIM, _SDIM)), full((1, _SDIM)),
          full((_NDIM, 1)), full((1, 1)),
          full((_NDIM, _SDIM)), full((1, _SDIM)),
          full((_NDIM, 1)), full((1, 1)),
      ],
      out_specs=[
          pl.BlockSpec((_G, _SDIM), lambda i: (0, 0)),
          pl.BlockSpec((_G, _SDIM), lambda i: (0, 0)),
      ],
      out_shape=[
          jax.ShapeDtypeStruct((_G, _SDIM), jnp.float32),
          jax.ShapeDtypeStruct((_G, _SDIM), jnp.float32),
      ],
  )(h, batch2, mfw, mfb, mgw, mgb, vfw, vfb, vgw, vgb)


def kernel(edge_index, node_atts, batch, params):
  p = params
  src = edge_index[0].astype(jnp.int32)
  dst = edge_index[1].astype(jnp.int32)

  npad = _EPAD - _E
  ar = jnp.arange(npad, dtype=jnp.int32)
  pad_g = (ar * 97) % _N
  pad_s = _N + (ar % 16)
  gidx = jnp.stack([jnp.concatenate([src, pad_g]),
                    jnp.concatenate([dst, pad_g])]).reshape(
                        2, _EPAD // 128, 128)
  sidx = jnp.stack([jnp.concatenate([dst, pad_s]),
                    jnp.concatenate([src, pad_s])]).reshape(
                        2, _EPAD // 128, 128)
  zeros_h = jnp.zeros((128, 128), jnp.float32)

  atts2 = node_atts.astype(jnp.int32).reshape(_N, 1)
  batch2 = batch.astype(jnp.int32).reshape(_N, 1)
  r2 = lambda b: b.reshape(1, -1)

  h = _embed(atts2, p['emb'])

  ones128_h = jnp.ones((128, 128), jnp.float32)
  out_d = _get_deg()(sidx, ones128_h, zeros_h)
  out_s = _get_spmm()(h[0], h[1], gidx, sidx, zeros_h)
  din = out_d[0, :_N, 0:1]
  dout = out_d[1, :_N, 0:1]
  h = _layer(out_s[0][:, :_N], out_s[1][:, :_N], h, din, dout,
             p['msg_W_0'], r2(p['msg_b_0']), p['msgr_W_0'], r2(p['msgr_b_0']),
             p['W_ih_0'], r2(p['b_ih_0']), p['W_hh_0'], r2(p['b_hh_0']))

  out_s2 = _get_spmm()(h[0], h[1], gidx, sidx, zeros_h)
  h = _layer(out_s2[0][:, :_N], out_s2[1][:, :_N], h, din, dout,
             p['msg_W_1'], r2(p['msg_b_1']), p['msgr_W_1'], r2(p['msgr_b_1']),
             p['W_ih_1'], r2(p['b_ih_1']), p['W_hh_1'], r2(p['b_hh_1']))

  return tuple(_pool(h, batch2,
                     p['mean_fm_W'], r2(p['mean_fm_b']),
                     p['mean_gm_W'], r2(p['mean_gm_b']),
                     p['var_fm_W'], r2(p['var_fm_b']),
                     p['var_gm_W'], r2(p['var_gm_b'])))


# speedup vs baseline: 11.0047x; 1.0028x over previous
import functools

import jax
import jax.numpy as jnp
from jax import lax
from jax.experimental import pallas as pl
from jax.experimental.pallas import tpu as pltpu
from jax.experimental.pallas import tpu_sc as plsc

_NDIM = 256
_SDIM = 256
_N = 10000
_E = 160000
_G = 64
_NTYPES = 11

_NC = 2
_NS = 16
_EPT = 10240
_EPAD = _EPT * _NS
_GROUP = 128
_NGROUPS = _EPT // _GROUP
_ACC_ROWS = 10112
_RPT = _ACC_ROWS // _NS
_WB_CHUNKS = (128, 128, 128, 128, 120)
_BLK = 1000
_NBLK = _N // _BLK


def _make_spmm():
  mesh = plsc.VectorSubcoreMesh(core_axis_name="c", subcore_axis_name="s",
                                num_cores=_NC, num_subcores=_NS)
  out_type = jax.ShapeDtypeStruct((_NC, 2, _ACC_ROWS, 128), jnp.float32)
  nstage = 8
  nchunks = (_EPT // 128) // nstage
  scratch = [
      pltpu.VMEM((2, 128, 128), jnp.float32),
      pltpu.VMEM((nstage, 128), jnp.int32),
      pltpu.VMEM((nstage, 128), jnp.int32),
      pltpu.VMEM_SHARED((_ACC_ROWS, 128), jnp.float32),
      pltpu.SemaphoreType.DMA,
      pltpu.SemaphoreType.DMA,
      pltpu.SemaphoreType.DMA,
      pltpu.SemaphoreType.DMA,
  ]

  def body(t0, t1, gidx_h, sidx_h, zeros_h, out_s, rows_v, gi_v, si_v, acc,
           sg0, sg1, ss0, ss1):
    cid = lax.axis_index("c")
    sid = lax.axis_index("s")

    for p in range(2):
      tab = t0 if p == 0 else t1
      pltpu.sync_copy(zeros_h, rows_v.at[0])
      off = 0
      for c in _WB_CHUNKS:
        pltpu.sync_copy(rows_v.at[0, pl.ds(0, c)],
                        acc.at[pl.ds(sid * _RPT + off, c)])
        off += c
      plsc.subcore_barrier()

      def pair(i, carry):
        d0 = pltpu.async_copy(tab.at[gi_v.at[2 * i]], rows_v.at[0], sg0)
        d1 = pltpu.async_copy(tab.at[gi_v.at[2 * i + 1]], rows_v.at[1], sg1)
        d0.wait()
        s0 = pltpu.async_copy(rows_v.at[0], acc.at[si_v.at[2 * i]], ss0,
                              add=True)
        d1.wait()
        s1 = pltpu.async_copy(rows_v.at[1], acc.at[si_v.at[2 * i + 1]], ss1,
                              add=True)
        s0.wait()
        s1.wait()
        return carry

      for chunk in range(nchunks):
        row0 = sid * (_EPT // 128) + chunk * nstage
        pltpu.sync_copy(gidx_h.at[cid, pl.ds(row0, nstage)], gi_v)
        pltpu.sync_copy(sidx_h.at[cid, pl.ds(row0, nstage)], si_v)
        lax.fori_loop(0, nstage // 2, pair, 0)
      plsc.subcore_barrier()

      off = 0
      for c in _WB_CHUNKS:
        sl = pl.ds(sid * _RPT + off, c)
        pltpu.sync_copy(acc.at[sl], rows_v.at[0, pl.ds(0, c)])
        pltpu.sync_copy(rows_v.at[0, pl.ds(0, c)], out_s.at[cid, p, sl])
        off += c

  return pl.kernel(body, out_type=out_type, mesh=mesh,
                   scratch_types=scratch)


def _make_deg():
  mesh = plsc.VectorSubcoreMesh(core_axis_name="c", subcore_axis_name="s",
                                num_cores=_NC, num_subcores=_NS)
  out_type = jax.ShapeDtypeStruct((_NC, _ACC_ROWS, 128), jnp.float32)
  scratch = [
      pltpu.VMEM((128, 128), jnp.float32),
      pltpu.VMEM((_EPT // 128, 128), jnp.int32),
      pltpu.VMEM_SHARED((_ACC_ROWS, 128), jnp.float32),
      pltpu.SemaphoreType.DMA,
  ]

  def body(sidx_h, ones_h, zeros_h, out_d, ones_v, si_v, dacc, sem):
    cid = lax.axis_index("c")
    sid = lax.axis_index("s")
    pltpu.sync_copy(sidx_h.at[cid, pl.ds(sid * (_EPT // 128), _EPT // 128)],
                    si_v)
    pltpu.sync_copy(zeros_h, ones_v)
    off = 0
    for c in _WB_CHUNKS:
      pltpu.sync_copy(ones_v.at[pl.ds(0, c)],
                      dacc.at[pl.ds(sid * _RPT + off, c)])
      off += c
    pltpu.sync_copy(ones_h, ones_v)
    plsc.subcore_barrier()

    def group(i, carry):
      ds = [pltpu.async_copy(ones_v, dacc.at[si_v.at[4 * i + j]], sem,
                             add=True) for j in range(4)]
      for d in ds:
        d.wait()
      return carry

    lax.fori_loop(0, _NGROUPS // 4, group, 0)
    plsc.subcore_barrier()

    off = 0
    for c in _WB_CHUNKS:
      sl = pl.ds(sid * _RPT + off, c)
      pltpu.sync_copy(dacc.at[sl], ones_v.at[pl.ds(0, c)])
      pltpu.sync_copy(ones_v.at[pl.ds(0, c)], out_d.at[cid, sl])
      off += c

  return pl.kernel(body, out_type=out_type, mesh=mesh,
                   scratch_types=scratch)


@functools.lru_cache(maxsize=None)
def _get_spmm():
  return _make_spmm()


@functools.lru_cache(maxsize=None)
def _get_deg():
  return _make_deg()


def _embed_body(atts_ref, emb_ref, out_ref):
  a = atts_ref[...]
  oh = (a == lax.broadcasted_iota(jnp.int32, (_BLK, _NTYPES), 1)
        ).astype(jnp.float32)
  h = jnp.dot(oh, emb_ref[...], preferred_element_type=jnp.float32)
  out_ref[0] = h[:, :128]
  out_ref[1] = h[:, 128:]


def _embed(atts2, emb):
  return pl.pallas_call(
      _embed_body,
      grid=(_NBLK,),
      in_specs=[
          pl.BlockSpec((_BLK, 1), lambda i: (i, 0)),
          pl.BlockSpec((_NTYPES, _NDIM), lambda i: (0, 0)),
      ],
      out_specs=pl.BlockSpec((2, _BLK, 128), lambda i: (0, i, 0)),
      out_shape=jax.ShapeDtypeStruct((2, _N, 128), jnp.float32),
  )(atts2, emb)


def _layer_body(sin_ref, sout_ref, h_ref, din_ref, dout_ref,
                wf_ref, bf_ref, wr_ref, br_ref,
                wih_ref, bih_ref, whh_ref, bhh_ref, out_ref):
  bf = jnp.bfloat16
  h = jnp.concatenate([h_ref[0], h_ref[1]], axis=1)
  hb = h.astype(bf)
  din = din_ref[...]
  dout = dout_ref[...]
  dot = functools.partial(jnp.dot, preferred_element_type=jnp.float32)
  aggr = (dot(sin_ref[0].astype(bf), wf_ref[0:128].astype(bf))
          + dot(sin_ref[1].astype(bf), wf_ref[128:256].astype(bf))
          + dot(sout_ref[0].astype(bf), wr_ref[0:128].astype(bf))
          + dot(sout_ref[1].astype(bf), wr_ref[128:256].astype(bf))
          + din * dot(hb, wf_ref[256:512].astype(bf))
          + dout * dot(hb, wr_ref[256:512].astype(bf))
          + din * bf_ref[...] + dout * br_ref[...])
  gi = dot(aggr.astype(bf), wih_ref[...].astype(bf)) + bih_ref[...]
  gh = dot(hb, whh_ref[...].astype(bf)) + bhh_ref[...]
  r = jax.nn.sigmoid(gi[:, 0:256] + gh[:, 0:256])
  z = jax.nn.sigmoid(gi[:, 256:512] + gh[:, 256:512])
  n = jnp.tanh(gi[:, 512:768] + r * gh[:, 512:768])
  hn = (1.0 - z) * n + z * h
  out_ref[0] = hn[:, 0:128]
  out_ref[1] = hn[:, 128:256]


def _layer(sin, sout, h, din, dout, wf, bf, wr, br, wih, bih, whh, bhh):
  full = lambda shape: pl.BlockSpec(shape, lambda i: tuple(0 for _ in shape))
  blk3 = pl.BlockSpec((2, _BLK, 128), lambda i: (0, i, 0))
  return pl.pallas_call(
      _layer_body,
      grid=(_NBLK,),
      in_specs=[
          blk3, blk3, blk3,
          pl.BlockSpec((_BLK, 1), lambda i: (i, 0)),
          pl.BlockSpec((_BLK, 1), lambda i: (i, 0)),
          full((2 * _NDIM, 2 * _NDIM)), full((1, 2 * _NDIM)),
          full((2 * _NDIM, 2 * _NDIM)), full((1, 2 * _NDIM)),
          full((2 * _NDIM, 3 * _NDIM)), full((1, 3 * _NDIM)),
          full((_NDIM, 3 * _NDIM)), full((1, 3 * _NDIM)),
      ],
      out_specs=pl.BlockSpec((2, _BLK, 128), lambda i: (0, i, 0)),
      out_shape=jax.ShapeDtypeStruct((2, _N, 128), jnp.float32),
  )(sin, sout, h, din, dout, wf, bf, wr, br, wih, bih, whh, bhh)


def _pool_body(h_ref, batch_ref,
               mfw_ref, mfb_ref, mgw_ref, mgb_ref,
               vfw_ref, vfb_ref, vgw_ref, vgb_ref,
               mout_ref, vout_ref):
  i = pl.program_id(0)

  @pl.when(i == 0)
  def _():
    mout_ref[...] = jnp.zeros_like(mout_ref)
    vout_ref[...] = jnp.zeros_like(vout_ref)

  h = jnp.concatenate([h_ref[0], h_ref[1]], axis=1)
  oh = (batch_ref[...] == lax.broadcasted_iota(jnp.int32, (_BLK, _G), 1)
        ).astype(jnp.float32)
  dot = functools.partial(jnp.dot, preferred_element_type=jnp.float32)
  for fw, fb, gw, gb, out in (
      (mfw_ref, mfb_ref, mgw_ref, mgb_ref, mout_ref),
      (vfw_ref, vfb_ref, vgw_ref, vgb_ref, vout_ref)):
    hv = dot(h, fw[...]) + fb[...]
    g = jax.nn.sigmoid(dot(h, gw[...]) + gb[...])
    out[...] += lax.dot_general(oh, hv * g, (((0,), (0,)), ((), ())),
                                preferred_element_type=jnp.float32)


def _pool(h, batch2, mfw, mfb, mgw, mgb, vfw, vfb, vgw, vgb):
  full = lambda shape: pl.BlockSpec(shape, lambda i: tuple(0 for _ in shape))
  return pl.pallas_call(
      _pool_body,
      grid=(_NBLK,),
      in_specs=[
          pl.BlockSpec((2, _BLK, 128), lambda i: (0, i, 0)),
          pl.BlockSpec((_BLK, 1), lambda i: (i, 0)),
          full((_NDIM, _SDIM)), full((1, _SDIM)),
          full((_NDIM, 1)), full((1, 1)),
          full((_NDIM, _SDIM)), full((1, _SDIM)),
          full((_NDIM, 1)), full((1, 1)),
      ],
      out_specs=[
          pl.BlockSpec((_G, _SDIM), lambda i: (0, 0)),
          pl.BlockSpec((_G, _SDIM), lambda i: (0, 0)),
      ],
      out_shape=[
          jax.ShapeDtypeStruct((_G, _SDIM), jnp.float32),
          jax.ShapeDtypeStruct((_G, _SDIM), jnp.float32),
      ],
  )(h, batch2, mfw, mfb, mgw, mgb, vfw, vfb, vgw, vgb)


def kernel(edge_index, node_atts, batch, params):
  p = params
  src = edge_index[0].astype(jnp.int32)
  dst = edge_index[1].astype(jnp.int32)

  npad = _EPAD - _E
  ar = jnp.arange(npad, dtype=jnp.int32)
  pad_g = (ar * 97) % _N
  pad_s = _N + (ar % 16)
  gidx = jnp.stack([jnp.concatenate([src, pad_g]),
                    jnp.concatenate([dst, pad_g])]).reshape(
                        2, _EPAD // 128, 128)
  sidx = jnp.stack([jnp.concatenate([dst, pad_s]),
                    jnp.concatenate([src, pad_s])]).reshape(
                        2, _EPAD // 128, 128)
  zeros_h = jnp.zeros((128, 128), jnp.float32)

  atts2 = node_atts.astype(jnp.int32).reshape(_N, 1)
  batch2 = batch.astype(jnp.int32).reshape(_N, 1)
  r2 = lambda b: b.reshape(1, -1)

  h = _embed(atts2, p['emb'])

  ones128_h = jnp.ones((128, 128), jnp.float32)
  out_d = _get_deg()(sidx, ones128_h, zeros_h)
  out_s = _get_spmm()(h[0], h[1], gidx, sidx, zeros_h)
  din = out_d[0, :_N, 0:1]
  dout = out_d[1, :_N, 0:1]
  h = _layer(out_s[0][:, :_N], out_s[1][:, :_N], h, din, dout,
             p['msg_W_0'], r2(p['msg_b_0']), p['msgr_W_0'], r2(p['msgr_b_0']),
             p['W_ih_0'], r2(p['b_ih_0']), p['W_hh_0'], r2(p['b_hh_0']))

  out_s2 = _get_spmm()(h[0], h[1], gidx, sidx, zeros_h)
  h = _layer(out_s2[0][:, :_N], out_s2[1][:, :_N], h, din, dout,
             p['msg_W_1'], r2(p['msg_b_1']), p['msgr_W_1'], r2(p['msgr_b_1']),
             p['W_ih_1'], r2(p['b_ih_1']), p['W_hh_1'], r2(p['b_hh_1']))

  return tuple(_pool(h, batch2,
                     p['mean_fm_W'], r2(p['mean_fm_b']),
                     p['mean_gm_W'], r2(p['mean_gm_b']),
                     p['var_fm_W'], r2(p['var_fm_b']),
                     p['var_gm_W'], r2(p['var_gm_b'])))


# speedup vs baseline: 11.4060x; 1.0365x over previous
import functools

import jax
import jax.numpy as jnp
from jax import lax
from jax.experimental import pallas as pl
from jax.experimental.pallas import tpu as pltpu
from jax.experimental.pallas import tpu_sc as plsc

_NDIM = 256
_SDIM = 256
_N = 10000
_E = 160000
_G = 64
_NTYPES = 11

_NC = 2
_NS = 16
_EPT = 10240
_EPAD = _EPT * _NS
_GROUP = 128
_NGROUPS = _EPT // _GROUP
_ACC_ROWS = 10112
_RPT = _ACC_ROWS // _NS
_WB_CHUNKS = (128, 128, 128, 128, 120)
_BLK = 1000
_NBLK = _N // _BLK


def _make_spmm():
  mesh = plsc.VectorSubcoreMesh(core_axis_name="c", subcore_axis_name="s",
                                num_cores=_NC, num_subcores=_NS)
  out_type = jax.ShapeDtypeStruct((_NC, 2, _ACC_ROWS, 128), jnp.float32)
  nstage = 8
  nchunks = (_EPT // 128) // nstage
  scratch = [
      pltpu.VMEM((2, 128, 128), jnp.float32),
      pltpu.VMEM((nstage, 128), jnp.int32),
      pltpu.VMEM((nstage, 128), jnp.int32),
      pltpu.VMEM_SHARED((_ACC_ROWS, 128), jnp.float32),
      pltpu.SemaphoreType.DMA,
      pltpu.SemaphoreType.DMA,
      pltpu.SemaphoreType.DMA,
      pltpu.SemaphoreType.DMA,
  ]

  def body(t0, t1, gidx_h, sidx_h, zeros_h, out_s, rows_v, gi_v, si_v, acc,
           sg0, sg1, ss0, ss1):
    cid = lax.axis_index("c")
    sid = lax.axis_index("s")

    for p in range(2):
      tab = t0 if p == 0 else t1
      pltpu.sync_copy(zeros_h, rows_v.at[0])
      off = 0
      for c in _WB_CHUNKS:
        pltpu.sync_copy(rows_v.at[0, pl.ds(0, c)],
                        acc.at[pl.ds(sid * _RPT + off, c)])
        off += c
      plsc.subcore_barrier()

      def pair(i, carry):
        d0 = pltpu.async_copy(tab.at[gi_v.at[2 * i]], rows_v.at[0], sg0)
        d1 = pltpu.async_copy(tab.at[gi_v.at[2 * i + 1]], rows_v.at[1], sg1)
        d0.wait()
        s0 = pltpu.async_copy(rows_v.at[0], acc.at[si_v.at[2 * i]], ss0,
                              add=True)
        d1.wait()
        s1 = pltpu.async_copy(rows_v.at[1], acc.at[si_v.at[2 * i + 1]], ss1,
                              add=True)
        s0.wait()
        s1.wait()
        return carry

      for chunk in range(nchunks):
        row0 = sid * (_EPT // 128) + chunk * nstage
        pltpu.sync_copy(gidx_h.at[cid, pl.ds(row0, nstage)], gi_v)
        pltpu.sync_copy(sidx_h.at[cid, pl.ds(row0, nstage)], si_v)
        lax.fori_loop(0, nstage // 2, pair, 0)
      plsc.subcore_barrier()

      off = 0
      for c in _WB_CHUNKS:
        sl = pl.ds(sid * _RPT + off, c)
        pltpu.sync_copy(acc.at[sl], rows_v.at[0, pl.ds(0, c)])
        pltpu.sync_copy(rows_v.at[0, pl.ds(0, c)], out_s.at[cid, p, sl])
        off += c

  return pl.kernel(body, out_type=out_type, mesh=mesh,
                   scratch_types=scratch)


def _make_deg():
  mesh = plsc.VectorSubcoreMesh(core_axis_name="c", subcore_axis_name="s",
                                num_cores=_NC, num_subcores=_NS)
  out_type = jax.ShapeDtypeStruct((_NC, _ACC_ROWS, 128), jnp.float32)
  scratch = [
      pltpu.VMEM((128, 128), jnp.float32),
      pltpu.VMEM((_EPT // 128, 128), jnp.int32),
      pltpu.VMEM_SHARED((_ACC_ROWS, 128), jnp.float32),
      pltpu.SemaphoreType.DMA,
  ]

  def body(sidx_h, ones_h, zeros_h, out_d, ones_v, si_v, dacc, sem):
    cid = lax.axis_index("c")
    sid = lax.axis_index("s")
    pltpu.sync_copy(sidx_h.at[cid, pl.ds(sid * (_EPT // 128), _EPT // 128)],
                    si_v)
    pltpu.sync_copy(zeros_h, ones_v)
    off = 0
    for c in _WB_CHUNKS:
      pltpu.sync_copy(ones_v.at[pl.ds(0, c)],
                      dacc.at[pl.ds(sid * _RPT + off, c)])
      off += c
    pltpu.sync_copy(ones_h, ones_v)
    plsc.subcore_barrier()

    def group(i, carry):
      ds = [pltpu.async_copy(ones_v, dacc.at[si_v.at[4 * i + j]], sem,
                             add=True) for j in range(4)]
      for d in ds:
        d.wait()
      return carry

    lax.fori_loop(0, _NGROUPS // 4, group, 0)
    plsc.subcore_barrier()

    off = 0
    for c in _WB_CHUNKS:
      sl = pl.ds(sid * _RPT + off, c)
      pltpu.sync_copy(dacc.at[sl], ones_v.at[pl.ds(0, c)])
      pltpu.sync_copy(ones_v.at[pl.ds(0, c)], out_d.at[cid, sl])
      off += c

  return pl.kernel(body, out_type=out_type, mesh=mesh,
                   scratch_types=scratch)


@functools.lru_cache(maxsize=None)
def _get_spmm():
  return _make_spmm()


@functools.lru_cache(maxsize=None)
def _get_deg():
  return _make_deg()


def _embed_body(atts_ref, emb_ref, out_ref):
  a = atts_ref[...]
  oh = (a == lax.broadcasted_iota(jnp.int32, (_BLK, _NTYPES), 1)
        ).astype(jnp.float32)
  h = jnp.dot(oh, emb_ref[...], preferred_element_type=jnp.float32)
  out_ref[0] = h[:, :128]
  out_ref[1] = h[:, 128:]


def _embed(atts2, emb):
  return pl.pallas_call(
      _embed_body,
      grid=(_NBLK,),
      in_specs=[
          pl.BlockSpec((_BLK, 1), lambda i: (i, 0)),
          pl.BlockSpec((_NTYPES, _NDIM), lambda i: (0, 0)),
      ],
      out_specs=pl.BlockSpec((2, _BLK, 128), lambda i: (0, i, 0)),
      out_shape=jax.ShapeDtypeStruct((2, _N, 128), jnp.float32),
  )(atts2, emb)


def _layer_body(s_ref, h_ref, d_ref,
                wf_ref, bf_ref, wr_ref, br_ref,
                wih_ref, bih_ref, whh_ref, bhh_ref, out_ref):
  bf = jnp.bfloat16
  h = jnp.concatenate([h_ref[0], h_ref[1]], axis=1)
  hb = h.astype(bf)
  din = d_ref[0][:, 0:1]
  dout = d_ref[1][:, 0:1]
  dot = functools.partial(jnp.dot, preferred_element_type=jnp.float32)
  aggr = (dot(s_ref[0, 0].astype(bf), wf_ref[0:128].astype(bf))
          + dot(s_ref[0, 1].astype(bf), wf_ref[128:256].astype(bf))
          + dot(s_ref[1, 0].astype(bf), wr_ref[0:128].astype(bf))
          + dot(s_ref[1, 1].astype(bf), wr_ref[128:256].astype(bf))
          + din * dot(hb, wf_ref[256:512].astype(bf))
          + dout * dot(hb, wr_ref[256:512].astype(bf))
          + din * bf_ref[...] + dout * br_ref[...])
  gi = dot(aggr.astype(bf), wih_ref[...].astype(bf)) + bih_ref[...]
  gh = dot(hb, whh_ref[...].astype(bf)) + bhh_ref[...]
  r = jax.nn.sigmoid(gi[:, 0:256] + gh[:, 0:256])
  z = jax.nn.sigmoid(gi[:, 256:512] + gh[:, 256:512])
  n = jnp.tanh(gi[:, 512:768] + r * gh[:, 512:768])
  hn = (1.0 - z) * n + z * h
  out_ref[0] = hn[:, 0:128]
  out_ref[1] = hn[:, 128:256]


def _layer(out_s, h, out_d, wf, bff, wr, br, wih, bih, whh, bhh):
  full = lambda shape: pl.BlockSpec(shape, lambda i: tuple(0 for _ in shape))
  return pl.pallas_call(
      _layer_body,
      grid=(_NBLK,),
      in_specs=[
          pl.BlockSpec((2, 2, _BLK, 128), lambda i: (0, 0, i, 0)),
          pl.BlockSpec((2, _BLK, 128), lambda i: (0, i, 0)),
          pl.BlockSpec((2, _BLK, 128), lambda i: (0, i, 0)),
          full((2 * _NDIM, 2 * _NDIM)), full((1, 2 * _NDIM)),
          full((2 * _NDIM, 2 * _NDIM)), full((1, 2 * _NDIM)),
          full((2 * _NDIM, 3 * _NDIM)), full((1, 3 * _NDIM)),
          full((_NDIM, 3 * _NDIM)), full((1, 3 * _NDIM)),
      ],
      out_specs=pl.BlockSpec((2, _BLK, 128), lambda i: (0, i, 0)),
      out_shape=jax.ShapeDtypeStruct((2, _N, 128), jnp.float32),
  )(out_s, h, out_d, wf, bff, wr, br, wih, bih, whh, bhh)


def _pool_body(h_ref, batch_ref,
               mfw_ref, mfb_ref, mgw_ref, mgb_ref,
               vfw_ref, vfb_ref, vgw_ref, vgb_ref,
               mout_ref, vout_ref):
  i = pl.program_id(0)

  @pl.when(i == 0)
  def _():
    mout_ref[...] = jnp.zeros_like(mout_ref)
    vout_ref[...] = jnp.zeros_like(vout_ref)

  h = jnp.concatenate([h_ref[0], h_ref[1]], axis=1)
  oh = (batch_ref[...] == lax.broadcasted_iota(jnp.int32, (_BLK, _G), 1)
        ).astype(jnp.float32)
  dot = functools.partial(jnp.dot, preferred_element_type=jnp.float32)
  for fw, fb, gw, gb, out in (
      (mfw_ref, mfb_ref, mgw_ref, mgb_ref, mout_ref),
      (vfw_ref, vfb_ref, vgw_ref, vgb_ref, vout_ref)):
    hv = dot(h, fw[...]) + fb[...]
    g = jax.nn.sigmoid(dot(h, gw[...]) + gb[...])
    out[...] += lax.dot_general(oh, hv * g, (((0,), (0,)), ((), ())),
                                preferred_element_type=jnp.float32)


def _pool(h, batch2, mfw, mfb, mgw, mgb, vfw, vfb, vgw, vgb):
  full = lambda shape: pl.BlockSpec(shape, lambda i: tuple(0 for _ in shape))
  return pl.pallas_call(
      _pool_body,
      grid=(_NBLK,),
      in_specs=[
          pl.BlockSpec((2, _BLK, 128), lambda i: (0, i, 0)),
          pl.BlockSpec((_BLK, 1), lambda i: (i, 0)),
          full((_NDIM, _SDIM)), full((1, _SDIM)),
          full((_NDIM, 1)), full((1, 1)),
          full((_NDIM, _SDIM)), full((1, _SDIM)),
          full((_NDIM, 1)), full((1, 1)),
      ],
      out_specs=[
          pl.BlockSpec((_G, _SDIM), lambda i: (0, 0)),
          pl.BlockSpec((_G, _SDIM), lambda i: (0, 0)),
      ],
      out_shape=[
          jax.ShapeDtypeStruct((_G, _SDIM), jnp.float32),
          jax.ShapeDtypeStruct((_G, _SDIM), jnp.float32),
      ],
  )(h, batch2, mfw, mfb, mgw, mgb, vfw, vfb, vgw, vgb)


def kernel(edge_index, node_atts, batch, params):
  p = params
  src = edge_index[0].astype(jnp.int32)
  dst = edge_index[1].astype(jnp.int32)

  npad = _EPAD - _E
  ar = jnp.arange(npad, dtype=jnp.int32)
  pad_g = (ar * 97) % _N
  pad_s = _N + (ar % 16)
  gidx = jnp.stack([jnp.concatenate([src, pad_g]),
                    jnp.concatenate([dst, pad_g])]).reshape(
                        2, _EPAD // 128, 128)
  sidx = jnp.stack([jnp.concatenate([dst, pad_s]),
                    jnp.concatenate([src, pad_s])]).reshape(
                        2, _EPAD // 128, 128)
  zeros_h = jnp.zeros((128, 128), jnp.float32)

  atts2 = node_atts.astype(jnp.int32).reshape(_N, 1)
  batch2 = batch.astype(jnp.int32).reshape(_N, 1)
  r2 = lambda b: b.reshape(1, -1)

  h = _embed(atts2, p['emb'])

  ones128_h = jnp.ones((128, 128), jnp.float32)
  out_d = _get_deg()(sidx, ones128_h, zeros_h)
  out_s = _get_spmm()(h[0], h[1], gidx, sidx, zeros_h)
  h = _layer(out_s, h, out_d,
             p['msg_W_0'], r2(p['msg_b_0']), p['msgr_W_0'], r2(p['msgr_b_0']),
             p['W_ih_0'], r2(p['b_ih_0']), p['W_hh_0'], r2(p['b_hh_0']))

  out_s2 = _get_spmm()(h[0], h[1], gidx, sidx, zeros_h)
  h = _layer(out_s2, h, out_d,
             p['msg_W_1'], r2(p['msg_b_1']), p['msgr_W_1'], r2(p['msgr_b_1']),
             p['W_ih_1'], r2(p['b_ih_1']), p['W_hh_1'], r2(p['b_hh_1']))

  return tuple(_pool(h, batch2,
                     p['mean_fm_W'], r2(p['mean_fm_b']),
                     p['mean_gm_W'], r2(p['mean_gm_b']),
                     p['var_fm_W'], r2(p['var_fm_b']),
                     p['var_gm_W'], r2(p['var_gm_b'])))


# speedup vs baseline: 11.5799x; 1.0152x over previous
import functools

import jax
import jax.numpy as jnp
from jax import lax
from jax.experimental import pallas as pl
from jax.experimental.pallas import tpu as pltpu
from jax.experimental.pallas import tpu_sc as plsc

_NDIM = 256
_SDIM = 256
_N = 10000
_E = 160000
_G = 64
_NTYPES = 11

_NC = 2
_NS = 16
_EPT = 10240
_EPAD = _EPT * _NS
_GROUP = 128
_NGROUPS = _EPT // _GROUP
_ACC_ROWS = 10112
_RPT = _ACC_ROWS // _NS
_WB_CHUNKS = (128, 128, 128, 128, 120)
_BLK = 1000
_NBLK = _N // _BLK


def _make_spmm():
  mesh = plsc.VectorSubcoreMesh(core_axis_name="c", subcore_axis_name="s",
                                num_cores=_NC, num_subcores=_NS)
  out_type = jax.ShapeDtypeStruct((_NC, 2, _ACC_ROWS, 128), jnp.float32)
  nstage = 8
  nchunks = (_EPT // 128) // nstage
  scratch = [
      pltpu.VMEM((2, 128, 128), jnp.float32),
      pltpu.VMEM((nstage, 128), jnp.int32),
      pltpu.VMEM((nstage, 128), jnp.int32),
      pltpu.VMEM_SHARED((_ACC_ROWS, 128), jnp.float32),
      pltpu.SemaphoreType.DMA,
      pltpu.SemaphoreType.DMA,
      pltpu.SemaphoreType.DMA,
      pltpu.SemaphoreType.DMA,
  ]

  def body(t_h, gidx_h, sidx_h, zeros_h, out_s, rows_v, gi_v, si_v, acc,
           sg0, sg1, ss0, ss1):
    cid = lax.axis_index("c")
    sid = lax.axis_index("s")

    for p in range(2):
      tab = t_h.at[p]
      pltpu.sync_copy(zeros_h, rows_v.at[0])
      off = 0
      for c in _WB_CHUNKS:
        pltpu.sync_copy(rows_v.at[0, pl.ds(0, c)],
                        acc.at[pl.ds(sid * _RPT + off, c)])
        off += c
      plsc.subcore_barrier()

      def pair(i, carry):
        d0 = pltpu.async_copy(tab.at[gi_v.at[2 * i]], rows_v.at[0], sg0)
        d1 = pltpu.async_copy(tab.at[gi_v.at[2 * i + 1]], rows_v.at[1], sg1)
        d0.wait()
        s0 = pltpu.async_copy(rows_v.at[0], acc.at[si_v.at[2 * i]], ss0,
                              add=True)
        d1.wait()
        s1 = pltpu.async_copy(rows_v.at[1], acc.at[si_v.at[2 * i + 1]], ss1,
                              add=True)
        s0.wait()
        s1.wait()
        return carry

      for chunk in range(nchunks):
        row0 = sid * (_EPT // 128) + chunk * nstage
        pltpu.sync_copy(gidx_h.at[cid, pl.ds(row0, nstage)], gi_v)
        pltpu.sync_copy(sidx_h.at[cid, pl.ds(row0, nstage)], si_v)
        lax.fori_loop(0, nstage // 2, pair, 0)
      plsc.subcore_barrier()

      off = 0
      for c in _WB_CHUNKS:
        sl = pl.ds(sid * _RPT + off, c)
        pltpu.sync_copy(acc.at[sl], rows_v.at[0, pl.ds(0, c)])
        pltpu.sync_copy(rows_v.at[0, pl.ds(0, c)], out_s.at[cid, p, sl])
        off += c

  return pl.kernel(body, out_type=out_type, mesh=mesh,
                   scratch_types=scratch)


def _make_deg():
  mesh = plsc.VectorSubcoreMesh(core_axis_name="c", subcore_axis_name="s",
                                num_cores=_NC, num_subcores=_NS)
  out_type = jax.ShapeDtypeStruct((_NC, _ACC_ROWS, 128), jnp.float32)
  scratch = [
      pltpu.VMEM((128, 128), jnp.float32),
      pltpu.VMEM((_EPT // 128, 128), jnp.int32),
      pltpu.VMEM_SHARED((_ACC_ROWS, 128), jnp.float32),
      pltpu.SemaphoreType.DMA,
  ]

  def body(sidx_h, ones_h, zeros_h, out_d, ones_v, si_v, dacc, sem):
    cid = lax.axis_index("c")
    sid = lax.axis_index("s")
    pltpu.sync_copy(sidx_h.at[cid, pl.ds(sid * (_EPT // 128), _EPT // 128)],
                    si_v)
    pltpu.sync_copy(zeros_h, ones_v)
    off = 0
    for c in _WB_CHUNKS:
      pltpu.sync_copy(ones_v.at[pl.ds(0, c)],
                      dacc.at[pl.ds(sid * _RPT + off, c)])
      off += c
    pltpu.sync_copy(ones_h, ones_v)
    plsc.subcore_barrier()

    def group(i, carry):
      ds = [pltpu.async_copy(ones_v, dacc.at[si_v.at[4 * i + j]], sem,
                             add=True) for j in range(4)]
      for d in ds:
        d.wait()
      return carry

    lax.fori_loop(0, _NGROUPS // 4, group, 0)
    plsc.subcore_barrier()

    off = 0
    for c in _WB_CHUNKS:
      sl = pl.ds(sid * _RPT + off, c)
      pltpu.sync_copy(dacc.at[sl], ones_v.at[pl.ds(0, c)])
      pltpu.sync_copy(ones_v.at[pl.ds(0, c)], out_d.at[cid, sl])
      off += c

  return pl.kernel(body, out_type=out_type, mesh=mesh,
                   scratch_types=scratch)


@functools.lru_cache(maxsize=None)
def _get_spmm():
  return _make_spmm()


@functools.lru_cache(maxsize=None)
def _get_deg():
  return _make_deg()


def _embed_body(atts_ref, emb_ref, out_ref):
  a = atts_ref[...]
  oh = (a == lax.broadcasted_iota(jnp.int32, (_BLK, _NTYPES), 1)
        ).astype(jnp.float32)
  h = jnp.dot(oh, emb_ref[...], preferred_element_type=jnp.float32)
  out_ref[0] = h[:, :128]
  out_ref[1] = h[:, 128:]


def _embed(atts2, emb):
  return pl.pallas_call(
      _embed_body,
      grid=(_NBLK,),
      in_specs=[
          pl.BlockSpec((_BLK, 1), lambda i: (i, 0)),
          pl.BlockSpec((_NTYPES, _NDIM), lambda i: (0, 0)),
      ],
      out_specs=pl.BlockSpec((2, _BLK, 128), lambda i: (0, i, 0)),
      out_shape=jax.ShapeDtypeStruct((2, _N, 128), jnp.float32),
  )(atts2, emb)


def _layer_body(s_ref, h_ref, d_ref,
                wf_ref, bf_ref, wr_ref, br_ref,
                wih_ref, bih_ref, whh_ref, bhh_ref, out_ref):
  bf = jnp.bfloat16
  h = jnp.concatenate([h_ref[0], h_ref[1]], axis=1)
  hb = h.astype(bf)
  din = d_ref[0][:, 0:1]
  dout = d_ref[1][:, 0:1]
  dot = functools.partial(jnp.dot, preferred_element_type=jnp.float32)
  aggr = (dot(s_ref[0, 0].astype(bf), wf_ref[0:128].astype(bf))
          + dot(s_ref[0, 1].astype(bf), wf_ref[128:256].astype(bf))
          + dot(s_ref[1, 0].astype(bf), wr_ref[0:128].astype(bf))
          + dot(s_ref[1, 1].astype(bf), wr_ref[128:256].astype(bf))
          + din * dot(hb, wf_ref[256:512].astype(bf))
          + dout * dot(hb, wr_ref[256:512].astype(bf))
          + din * bf_ref[...] + dout * br_ref[...])
  gi = dot(aggr.astype(bf), wih_ref[...].astype(bf)) + bih_ref[...]
  gh = dot(hb, whh_ref[...].astype(bf)) + bhh_ref[...]
  r = jax.nn.sigmoid(gi[:, 0:256] + gh[:, 0:256])
  z = jax.nn.sigmoid(gi[:, 256:512] + gh[:, 256:512])
  n = jnp.tanh(gi[:, 512:768] + r * gh[:, 512:768])
  hn = (1.0 - z) * n + z * h
  out_ref[0] = hn[:, 0:128]
  out_ref[1] = hn[:, 128:256]


def _layer(out_s, h, out_d, wf, bff, wr, br, wih, bih, whh, bhh):
  full = lambda shape: pl.BlockSpec(shape, lambda i: tuple(0 for _ in shape))
  return pl.pallas_call(
      _layer_body,
      grid=(_NBLK,),
      in_specs=[
          pl.BlockSpec((2, 2, _BLK, 128), lambda i: (0, 0, i, 0)),
          pl.BlockSpec((2, _BLK, 128), lambda i: (0, i, 0)),
          pl.BlockSpec((2, _BLK, 128), lambda i: (0, i, 0)),
          full((2 * _NDIM, 2 * _NDIM)), full((1, 2 * _NDIM)),
          full((2 * _NDIM, 2 * _NDIM)), full((1, 2 * _NDIM)),
          full((2 * _NDIM, 3 * _NDIM)), full((1, 3 * _NDIM)),
          full((_NDIM, 3 * _NDIM)), full((1, 3 * _NDIM)),
      ],
      out_specs=pl.BlockSpec((2, _BLK, 128), lambda i: (0, i, 0)),
      out_shape=jax.ShapeDtypeStruct((2, _N, 128), jnp.float32),
  )(out_s, h, out_d, wf, bff, wr, br, wih, bih, whh, bhh)


def _pool_body(h_ref, batch_ref,
               mfw_ref, mfb_ref, mgw_ref, mgb_ref,
               vfw_ref, vfb_ref, vgw_ref, vgb_ref,
               mout_ref, vout_ref):
  i = pl.program_id(0)

  @pl.when(i == 0)
  def _():
    mout_ref[...] = jnp.zeros_like(mout_ref)
    vout_ref[...] = jnp.zeros_like(vout_ref)

  h = jnp.concatenate([h_ref[0], h_ref[1]], axis=1)
  oh = (batch_ref[...] == lax.broadcasted_iota(jnp.int32, (_BLK, _G), 1)
        ).astype(jnp.float32)
  dot = functools.partial(jnp.dot, preferred_element_type=jnp.float32)
  for fw, fb, gw, gb, out in (
      (mfw_ref, mfb_ref, mgw_ref, mgb_ref, mout_ref),
      (vfw_ref, vfb_ref, vgw_ref, vgb_ref, vout_ref)):
    hv = dot(h, fw[...]) + fb[...]
    g = jax.nn.sigmoid(dot(h, gw[...]) + gb[...])
    out[...] += lax.dot_general(oh, hv * g, (((0,), (0,)), ((), ())),
                                preferred_element_type=jnp.float32)


def _pool(h, batch2, mfw, mfb, mgw, mgb, vfw, vfb, vgw, vgb):
  full = lambda shape: pl.BlockSpec(shape, lambda i: tuple(0 for _ in shape))
  return pl.pallas_call(
      _pool_body,
      grid=(_NBLK,),
      in_specs=[
          pl.BlockSpec((2, _BLK, 128), lambda i: (0, i, 0)),
          pl.BlockSpec((_BLK, 1), lambda i: (i, 0)),
          full((_NDIM, _SDIM)), full((1, _SDIM)),
          full((_NDIM, 1)), full((1, 1)),
          full((_NDIM, _SDIM)), full((1, _SDIM)),
          full((_NDIM, 1)), full((1, 1)),
      ],
      out_specs=[
          pl.BlockSpec((_G, _SDIM), lambda i: (0, 0)),
          pl.BlockSpec((_G, _SDIM), lambda i: (0, 0)),
      ],
      out_shape=[
          jax.ShapeDtypeStruct((_G, _SDIM), jnp.float32),
          jax.ShapeDtypeStruct((_G, _SDIM), jnp.float32),
      ],
  )(h, batch2, mfw, mfb, mgw, mgb, vfw, vfb, vgw, vgb)


def kernel(edge_index, node_atts, batch, params):
  p = params
  src = edge_index[0].astype(jnp.int32)
  dst = edge_index[1].astype(jnp.int32)

  npad = _EPAD - _E
  ar = jnp.arange(npad, dtype=jnp.int32)
  pad_g = (ar * 97) % _N
  pad_s = _N + (ar % 16)
  gidx = jnp.stack([jnp.concatenate([src, pad_g]),
                    jnp.concatenate([dst, pad_g])]).reshape(
                        2, _EPAD // 128, 128)
  sidx = jnp.stack([jnp.concatenate([dst, pad_s]),
                    jnp.concatenate([src, pad_s])]).reshape(
                        2, _EPAD // 128, 128)
  zeros_h = jnp.zeros((128, 128), jnp.float32)

  atts2 = node_atts.astype(jnp.int32).reshape(_N, 1)
  batch2 = batch.astype(jnp.int32).reshape(_N, 1)
  r2 = lambda b: b.reshape(1, -1)

  h = _embed(atts2, p['emb'])

  ones128_h = jnp.ones((128, 128), jnp.float32)
  out_d = _get_deg()(sidx, ones128_h, zeros_h)
  out_s = _get_spmm()(h, gidx, sidx, zeros_h)
  h = _layer(out_s, h, out_d,
             p['msg_W_0'], r2(p['msg_b_0']), p['msgr_W_0'], r2(p['msgr_b_0']),
             p['W_ih_0'], r2(p['b_ih_0']), p['W_hh_0'], r2(p['b_hh_0']))

  out_s2 = _get_spmm()(h, gidx, sidx, zeros_h)
  h = _layer(out_s2, h, out_d,
             p['msg_W_1'], r2(p['msg_b_1']), p['msgr_W_1'], r2(p['msgr_b_1']),
             p['W_ih_1'], r2(p['b_ih_1']), p['W_hh_1'], r2(p['b_hh_1']))

  return tuple(_pool(h, batch2,
                     p['mean_fm_W'], r2(p['mean_fm_b']),
                     p['mean_gm_W'], r2(p['mean_gm_b']),
                     p['var_fm_W'], r2(p['var_fm_b']),
                     p['var_gm_W'], r2(p['var_gm_b'])))


# speedup vs baseline: 11.5951x; 1.0013x over previous
import functools

import jax
import jax.numpy as jnp
from jax import lax
from jax.experimental import pallas as pl
from jax.experimental.pallas import tpu as pltpu
from jax.experimental.pallas import tpu_sc as plsc

_NDIM = 256
_SDIM = 256
_N = 10000
_E = 160000
_G = 64
_NTYPES = 11

_NC = 2
_NS = 16
_EPT = 10240
_EPAD = _EPT * _NS
_GROUP = 128
_NGROUPS = _EPT // _GROUP
_ACC_ROWS = 10112
_RPT = _ACC_ROWS // _NS
_WB_CHUNKS = (128, 128, 128, 128, 120)
_BLK = 1000
_NBLK = _N // _BLK


def _make_spmm(with_deg):
  mesh = plsc.VectorSubcoreMesh(core_axis_name="c", subcore_axis_name="s",
                                num_cores=_NC, num_subcores=_NS)
  out_type = [jax.ShapeDtypeStruct((_NC, 2, _ACC_ROWS, 128), jnp.float32)]
  if with_deg:
    out_type.append(jax.ShapeDtypeStruct((_NC, _ACC_ROWS, 128), jnp.float32))
  nstage = 8
  nchunks = (_EPT // 128) // nstage
  scratch = [
      pltpu.VMEM((2, 128, 128), jnp.float32),
      pltpu.VMEM((nstage, 128), jnp.int32),
      pltpu.VMEM((nstage, 128), jnp.int32),
      pltpu.VMEM_SHARED((_ACC_ROWS, 128), jnp.float32),
      pltpu.SemaphoreType.DMA,
      pltpu.SemaphoreType.DMA,
      pltpu.SemaphoreType.DMA,
      pltpu.SemaphoreType.DMA,
  ]

  def body(*refs):
    if with_deg:
      (t_h, gidx_h, sidx_h, zeros_h, ones_h, out_s, out_d,
       rows_v, gi_v, si_v, acc, sg0, sg1, ss0, ss1) = refs
    else:
      (t_h, gidx_h, sidx_h, zeros_h, out_s,
       rows_v, gi_v, si_v, acc, sg0, sg1, ss0, ss1) = refs
    cid = lax.axis_index("c")
    sid = lax.axis_index("s")

    def zero_acc():
      pltpu.sync_copy(zeros_h, rows_v.at[0])
      off = 0
      for c in _WB_CHUNKS:
        pltpu.sync_copy(rows_v.at[0, pl.ds(0, c)],
                        acc.at[pl.ds(sid * _RPT + off, c)])
        off += c
      plsc.subcore_barrier()

    def writeback(dst):
      off = 0
      for c in _WB_CHUNKS:
        sl = pl.ds(sid * _RPT + off, c)
        pltpu.sync_copy(acc.at[sl], rows_v.at[0, pl.ds(0, c)])
        pltpu.sync_copy(rows_v.at[0, pl.ds(0, c)], dst.at[sl])
        off += c

    if with_deg:
      zero_acc()
      pltpu.sync_copy(ones_h, rows_v.at[1])

      def dgroup(i, carry):
        ds = [pltpu.async_copy(rows_v.at[1], acc.at[si_v.at[4 * i + j]],
                               ss0, add=True) for j in range(4)]
        for d in ds:
          d.wait()
        return carry

      for chunk in range(nchunks):
        row0 = sid * (_EPT // 128) + chunk * nstage
        pltpu.sync_copy(sidx_h.at[cid, pl.ds(row0, nstage)], si_v)
        lax.fori_loop(0, nstage // 4, dgroup, 0)
      plsc.subcore_barrier()
      writeback(out_d.at[cid])

    for p in range(2):
      tab = t_h.at[p]
      zero_acc()

      def pair(i, carry):
        d0 = pltpu.async_copy(tab.at[gi_v.at[2 * i]], rows_v.at[0], sg0)
        d1 = pltpu.async_copy(tab.at[gi_v.at[2 * i + 1]], rows_v.at[1], sg1)
        d0.wait()
        s0 = pltpu.async_copy(rows_v.at[0], acc.at[si_v.at[2 * i]], ss0,
                              add=True)
        d1.wait()
        s1 = pltpu.async_copy(rows_v.at[1], acc.at[si_v.at[2 * i + 1]], ss1,
                              add=True)
        s0.wait()
        s1.wait()
        return carry

      for chunk in range(nchunks):
        row0 = sid * (_EPT // 128) + chunk * nstage
        pltpu.sync_copy(gidx_h.at[cid, pl.ds(row0, nstage)], gi_v)
        pltpu.sync_copy(sidx_h.at[cid, pl.ds(row0, nstage)], si_v)
        lax.fori_loop(0, nstage // 2, pair, 0)
      plsc.subcore_barrier()
      writeback(out_s.at[cid, p])

  return pl.kernel(body, out_type=tuple(out_type), mesh=mesh,
                   scratch_types=scratch)


@functools.lru_cache(maxsize=None)
def _get_spmm(with_deg):
  return _make_spmm(with_deg)


def _embed_body(atts_ref, emb_ref, out_ref):
  a = atts_ref[...]
  oh = (a == lax.broadcasted_iota(jnp.int32, (_BLK, _NTYPES), 1)
        ).astype(jnp.float32)
  h = jnp.dot(oh, emb_ref[...], preferred_element_type=jnp.float32)
  out_ref[0] = h[:, :128]
  out_ref[1] = h[:, 128:]


def _embed(atts2, emb):
  return pl.pallas_call(
      _embed_body,
      grid=(_NBLK,),
      in_specs=[
          pl.BlockSpec((_BLK, 1), lambda i: (i, 0)),
          pl.BlockSpec((_NTYPES, _NDIM), lambda i: (0, 0)),
      ],
      out_specs=pl.BlockSpec((2, _BLK, 128), lambda i: (0, i, 0)),
      out_shape=jax.ShapeDtypeStruct((2, _N, 128), jnp.float32),
  )(atts2, emb)


def _layer_body(s_ref, h_ref, d_ref,
                wf_ref, bf_ref, wr_ref, br_ref,
                wih_ref, bih_ref, whh_ref, bhh_ref, out_ref):
  bf = jnp.bfloat16
  h = jnp.concatenate([h_ref[0], h_ref[1]], axis=1)
  hb = h.astype(bf)
  din = d_ref[0][:, 0:1]
  dout = d_ref[1][:, 0:1]
  dot = functools.partial(jnp.dot, preferred_element_type=jnp.float32)
  aggr = (dot(s_ref[0, 0].astype(bf), wf_ref[0:128].astype(bf))
          + dot(s_ref[0, 1].astype(bf), wf_ref[128:256].astype(bf))
          + dot(s_ref[1, 0].astype(bf), wr_ref[0:128].astype(bf))
          + dot(s_ref[1, 1].astype(bf), wr_ref[128:256].astype(bf))
          + din * dot(hb, wf_ref[256:512].astype(bf))
          + dout * dot(hb, wr_ref[256:512].astype(bf))
          + din * bf_ref[...] + dout * br_ref[...])
  gi = dot(aggr.astype(bf), wih_ref[...].astype(bf)) + bih_ref[...]
  gh = dot(hb, whh_ref[...].astype(bf)) + bhh_ref[...]
  r = jax.nn.sigmoid(gi[:, 0:256] + gh[:, 0:256])
  z = jax.nn.sigmoid(gi[:, 256:512] + gh[:, 256:512])
  n = jnp.tanh(gi[:, 512:768] + r * gh[:, 512:768])
  hn = (1.0 - z) * n + z * h
  out_ref[0] = hn[:, 0:128]
  out_ref[1] = hn[:, 128:256]


def _layer(out_s, h, out_d, wf, bff, wr, br, wih, bih, whh, bhh):
  full = lambda shape: pl.BlockSpec(shape, lambda i: tuple(0 for _ in shape))
  return pl.pallas_call(
      _layer_body,
      grid=(_NBLK,),
      in_specs=[
          pl.BlockSpec((2, 2, _BLK, 128), lambda i: (0, 0, i, 0)),
          pl.BlockSpec((2, _BLK, 128), lambda i: (0, i, 0)),
          pl.BlockSpec((2, _BLK, 128), lambda i: (0, i, 0)),
          full((2 * _NDIM, 2 * _NDIM)), full((1, 2 * _NDIM)),
          full((2 * _NDIM, 2 * _NDIM)), full((1, 2 * _NDIM)),
          full((2 * _NDIM, 3 * _NDIM)), full((1, 3 * _NDIM)),
          full((_NDIM, 3 * _NDIM)), full((1, 3 * _NDIM)),
      ],
      out_specs=pl.BlockSpec((2, _BLK, 128), lambda i: (0, i, 0)),
      out_shape=jax.ShapeDtypeStruct((2, _N, 128), jnp.float32),
  )(out_s, h, out_d, wf, bff, wr, br, wih, bih, whh, bhh)


def _pool_body(h_ref, batch_ref,
               mfw_ref, mfb_ref, mgw_ref, mgb_ref,
               vfw_ref, vfb_ref, vgw_ref, vgb_ref,
               mout_ref, vout_ref):
  i = pl.program_id(0)

  @pl.when(i == 0)
  def _():
    mout_ref[...] = jnp.zeros_like(mout_ref)
    vout_ref[...] = jnp.zeros_like(vout_ref)

  h = jnp.concatenate([h_ref[0], h_ref[1]], axis=1)
  oh = (batch_ref[...] == lax.broadcasted_iota(jnp.int32, (_BLK, _G), 1)
        ).astype(jnp.float32)
  dot = functools.partial(jnp.dot, preferred_element_type=jnp.float32)
  for fw, fb, gw, gb, out in (
      (mfw_ref, mfb_ref, mgw_ref, mgb_ref, mout_ref),
      (vfw_ref, vfb_ref, vgw_ref, vgb_ref, vout_ref)):
    hv = dot(h, fw[...]) + fb[...]
    g = jax.nn.sigmoid(dot(h, gw[...]) + gb[...])
    out[...] += lax.dot_general(oh, hv * g, (((0,), (0,)), ((), ())),
                                preferred_element_type=jnp.float32)


def _pool(h, batch2, mfw, mfb, mgw, mgb, vfw, vfb, vgw, vgb):
  full = lambda shape: pl.BlockSpec(shape, lambda i: tuple(0 for _ in shape))
  return pl.pallas_call(
      _pool_body,
      grid=(_NBLK,),
      in_specs=[
          pl.BlockSpec((2, _BLK, 128), lambda i: (0, i, 0)),
          pl.BlockSpec((_BLK, 1), lambda i: (i, 0)),
          full((_NDIM, _SDIM)), full((1, _SDIM)),
          full((_NDIM, 1)), full((1, 1)),
          full((_NDIM, _SDIM)), full((1, _SDIM)),
          full((_NDIM, 1)), full((1, 1)),
      ],
      out_specs=[
          pl.BlockSpec((_G, _SDIM), lambda i: (0, 0)),
          pl.BlockSpec((_G, _SDIM), lambda i: (0, 0)),
      ],
      out_shape=[
          jax.ShapeDtypeStruct((_G, _SDIM), jnp.float32),
          jax.ShapeDtypeStruct((_G, _SDIM), jnp.float32),
      ],
  )(h, batch2, mfw, mfb, mgw, mgb, vfw, vfb, vgw, vgb)


def kernel(edge_index, node_atts, batch, params):
  p = params
  src = edge_index[0].astype(jnp.int32)
  dst = edge_index[1].astype(jnp.int32)

  npad = _EPAD - _E
  ar = jnp.arange(npad, dtype=jnp.int32)
  pad_g = (ar * 97) % _N
  pad_s = _N + (ar % 16)
  gidx = jnp.stack([jnp.concatenate([src, pad_g]),
                    jnp.concatenate([dst, pad_g])]).reshape(
                        2, _EPAD // 128, 128)
  sidx = jnp.stack([jnp.concatenate([dst, pad_s]),
                    jnp.concatenate([src, pad_s])]).reshape(
                        2, _EPAD // 128, 128)
  zeros_h = jnp.zeros((128, 128), jnp.float32)

  atts2 = node_atts.astype(jnp.int32).reshape(_N, 1)
  batch2 = batch.astype(jnp.int32).reshape(_N, 1)
  r2 = lambda b: b.reshape(1, -1)

  h = _embed(atts2, p['emb'])

  ones128_h = jnp.ones((128, 128), jnp.float32)
  out_s, out_d = _get_spmm(True)(h, gidx, sidx, zeros_h, ones128_h)
  h = _layer(out_s, h, out_d,
             p['msg_W_0'], r2(p['msg_b_0']), p['msgr_W_0'], r2(p['msgr_b_0']),
             p['W_ih_0'], r2(p['b_ih_0']), p['W_hh_0'], r2(p['b_hh_0']))

  (out_s2,) = _get_spmm(False)(h, gidx, sidx, zeros_h)
  h = _layer(out_s2, h, out_d,
             p['msg_W_1'], r2(p['msg_b_1']), p['msgr_W_1'], r2(p['msgr_b_1']),
             p['W_ih_1'], r2(p['b_ih_1']), p['W_hh_1'], r2(p['b_hh_1']))

  return tuple(_pool(h, batch2,
                     p['mean_fm_W'], r2(p['mean_fm_b']),
                     p['mean_gm_W'], r2(p['mean_gm_b']),
                     p['var_fm_W'], r2(p['var_fm_b']),
                     p['var_gm_W'], r2(p['var_gm_b'])))
